# baseline (speedup 1.0000x reference)
import functools

import jax
import jax.numpy as jnp
from jax import lax
from jax.experimental import pallas as pl
from jax.experimental.pallas import tpu as pltpu, tpu_sc as plsc

NC, NS, L = 2, 16, 16
NW = NC * NS
C = 128
K = 4

_SC_PARAMS = pltpu.CompilerParams(
    needs_layout_passes=False, use_tc_tiling_on_sc=False
)


def _sc_deg(col_r, w_r, n_pad):
    T = col_r.shape[1]
    rows_per_tile = n_pad // NS
    mesh = plsc.VectorSubcoreMesh(core_axis_name="c", subcore_axis_name="s")

    @functools.partial(
        pl.kernel,
        mesh=mesh,
        out_type=jax.ShapeDtypeStruct((NC, n_pad), jnp.float32),
        scratch_types=[
            pltpu.VMEM((T, C), jnp.int32),
            pltpu.VMEM((T, C), jnp.float32),
            pltpu.VMEM((rows_per_tile,), jnp.float32),
            pltpu.VMEM_SHARED((n_pad,), jnp.float32),
            pltpu.SemaphoreType.DMA,
        ],
        compiler_params=_SC_PARAMS,
    )
    def k(col_hbm, w_hbm, out_hbm, col_v, w_v, zbuf, acc_sh, sem):
        c = lax.axis_index("c")
        s = lax.axis_index("s")
        wid = s * NC + c
        pltpu.sync_copy(col_hbm.at[wid], col_v)
        pltpu.sync_copy(w_hbm.at[wid], w_v)

        @pl.loop(0, rows_per_tile // L)
        def _zero(i):
            zbuf[pl.ds(i * L, L)] = jnp.zeros((L,), jnp.float32)

        pltpu.sync_copy(zbuf, acc_sh.at[pl.ds(s * rows_per_tile, rows_per_tile)])
        plsc.subcore_barrier()

        @pl.loop(0, T)
        def _fire(j):
            pltpu.async_copy(w_v.at[j], acc_sh.at[col_v.at[j]], sem, add=True)

        @pl.loop(0, T)
        def _drain(j):
            pltpu.make_async_copy(w_v.at[j], acc_sh.at[col_v.at[j]], sem).wait()

        plsc.subcore_barrier()
        pltpu.sync_copy(
            acc_sh.at[pl.ds(s * rows_per_tile, rows_per_tile)],
            out_hbm.at[c, pl.ds(s * rows_per_tile, rows_per_tile)],
        )

    return k(col_r, w_r)


def _sc_agg(hd, row_r, col_r, w_r, d_pad, n_pad):
    T = row_r.shape[1]
    rows_per_tile = n_pad // NS
    nz = rows_per_tile // C
    R = 2 * K
    mesh = plsc.VectorSubcoreMesh(core_axis_name="c", subcore_axis_name="s")

    @functools.partial(
        pl.kernel,
        mesh=mesh,
        out_type=jax.ShapeDtypeStruct((NC, n_pad, d_pad), jnp.float32),
        scratch_types=[
            pltpu.VMEM((T, C), jnp.int32),
            pltpu.VMEM((T, C), jnp.int32),
            pltpu.VMEM((T, C), jnp.float32),
            pltpu.VMEM((R, C, d_pad), jnp.float32),
            pltpu.VMEM_SHARED((n_pad, d_pad), jnp.float32),
        ] + [pltpu.SemaphoreType.DMA] * (2 * R),
        compiler_params=_SC_PARAMS,
    )
    def k(hd_hbm, row_hbm, col_hbm, w_hbm, out_hbm,
          row_v, col_v, w_v, msg_v, acc_sh, *sems):
        gsem = sems[:R]
        ssem = sems[R:]
        c = lax.axis_index("c")
        s = lax.axis_index("s")
        wid = s * NC + c
        pltpu.sync_copy(row_hbm.at[wid], row_v)
        pltpu.sync_copy(col_hbm.at[wid], col_v)
        pltpu.sync_copy(w_hbm.at[wid], w_v)

        zb = msg_v.at[0]

        @pl.loop(0, (C * d_pad) // L)
        def _zero(i):
            r = i // (d_pad // L)
            kk = i % (d_pad // L)
            zb[r, pl.ds(kk * L, L)] = jnp.zeros((L,), jnp.float32)

        @pl.loop(0, nz)
        def _zcopy(kz):
            pltpu.sync_copy(zb, acc_sh.at[pl.ds(s * rows_per_tile + kz * C, C)])

        plsc.subcore_barrier()

        for b in range(K):
            pltpu.async_copy(hd_hbm.at[row_v.at[b]], msg_v.at[b], gsem[b])

        def scale_chunk(mb, j):
            @pl.loop(0, C // 16)
            def _grp(q):
                wrow = w_v[j, pl.ds(q * 16, 16)]
                for l in range(16):
                    wv = jnp.full((L,), wrow[l], jnp.float32)
                    e = q * 16 + l
                    for kk in range(d_pad // L):
                        sl = pl.ds(kk * L, L)
                        mb[e, sl] = mb[e, sl] * wv

        @pl.loop(0, T // R)
        def _ring(gi):
            for v in range(R):
                j = gi * R + v
                mb = msg_v.at[v]
                pltpu.make_async_copy(hd_hbm.at[row_v.at[j]], mb, gsem[v]).wait()
                scale_chunk(mb, j)
                pltpu.async_copy(mb, acc_sh.at[col_v.at[j]], ssem[v], add=True)

                jg = j + K
                bg = (v + K) % R
                mg = msg_v.at[bg]

                @pl.when(jg < T)
                def _refill():
                    @pl.when(jg >= R)
                    def _drain_prev_scatter():
                        pltpu.make_async_copy(
                            mg, acc_sh.at[col_v.at[jg - R]], ssem[bg]
                        ).wait()

                    pltpu.async_copy(hd_hbm.at[row_v.at[jg]], mg, gsem[bg])

        for b in range(R):
            pltpu.make_async_copy(
                msg_v.at[b], acc_sh.at[col_v.at[T - R + b]], ssem[b]
            ).wait()

        plsc.subcore_barrier()
        pltpu.sync_copy(
            acc_sh.at[pl.ds(s * rows_per_tile, rows_per_tile)],
            out_hbm.at[c, pl.ds(s * rows_per_tile, rows_per_tile)],
        )

    return k(hd, row_r, col_r, w_r)


def _pad_d(x, d_pad):
    d = x.shape[1]
    if d == d_pad:
        return x
    return jnp.pad(x, ((0, 0), (0, d_pad - d)))


_RB = 1000


def _row_spec(d):
    return pl.BlockSpec((_RB, d), lambda r: (r, 0))


def _full_spec(a, b):
    return pl.BlockSpec((a, b), lambda r: (0, 0))


def _tc_pre1(x, Wp, p0, p1):
    n, d_in = x.shape
    d_o = Wp.shape[1]

    def body(x_ref, w_ref, p0_ref, p1_ref, hd_ref, h_ref, dinv_ref):
        dinv = lax.rsqrt(1.0 + p0_ref[...] + p1_ref[...])
        h = jnp.dot(x_ref[...], w_ref[...], preferred_element_type=jnp.float32)
        h_ref[...] = h
        hd_ref[...] = h * dinv
        dinv_ref[...] = dinv

    return pl.pallas_call(
        body,
        grid=(n // _RB,),
        in_specs=[_row_spec(d_in), _full_spec(d_in, d_o),
                  _row_spec(1), _row_spec(1)],
        out_specs=[_row_spec(d_o), _row_spec(d_o), _row_spec(1)],
        out_shape=[jax.ShapeDtypeStruct((n, d_o), jnp.float32),
                   jax.ShapeDtypeStruct((n, d_o), jnp.float32),
                   jax.ShapeDtypeStruct((n, 1), jnp.float32)],
    )(x, Wp, p0, p1)


def _tc_step(P0, P1, h, dinv, bp, Wp):
    n, d_prev = h.shape
    d_next = Wp.shape[1]

    def body(p0_ref, p1_ref, h_ref, dinv_ref, b_ref, w_ref, hn_ref, hdn_ref):
        dinv = dinv_ref[...]
        s = p0_ref[...] + p1_ref[...]
        out = dinv * s + (dinv * dinv) * h_ref[...] + b_ref[...]
        xn = jnp.maximum(out, 0.0)
        hn = jnp.dot(xn, w_ref[...], preferred_element_type=jnp.float32)
        hn_ref[...] = hn
        hdn_ref[...] = hn * dinv

    return pl.pallas_call(
        body,
        grid=(n // _RB,),
        in_specs=[_row_spec(d_prev), _row_spec(d_prev), _row_spec(d_prev),
                  _row_spec(1), _full_spec(1, d_prev),
                  _full_spec(d_prev, d_next)],
        out_specs=[_row_spec(d_next), _row_spec(d_next)],
        out_shape=[jax.ShapeDtypeStruct((n, d_next), jnp.float32),
                   jax.ShapeDtypeStruct((n, d_next), jnp.float32)],
    )(P0, P1, h, dinv, bp, Wp)


def _tc_final(P0, P1, x5, dinv, Wp, bp, d_out):
    n, d_prev = x5.shape
    d_op = Wp.shape[1]

    def body(p0_ref, p1_ref, x_ref, dinv_ref, w_ref, b_ref, o_ref):
        dinv = dinv_ref[...]
        s = p0_ref[...] + p1_ref[...]
        ax = dinv * s + (dinv * dinv) * x_ref[...]
        out = jnp.dot(ax, w_ref[...], preferred_element_type=jnp.float32)
        out = out[:, :d_out] + b_ref[...]
        m = jnp.max(out, axis=1, keepdims=True)
        z = out - m
        lse = jnp.log(jnp.sum(jnp.exp(z), axis=1, keepdims=True))
        o_ref[...] = z - lse

    return pl.pallas_call(
        body,
        grid=(n // _RB,),
        in_specs=[_row_spec(d_prev), _row_spec(d_prev), _row_spec(d_prev),
                  _row_spec(1), _full_spec(d_prev, d_op),
                  _full_spec(1, d_out)],
        out_specs=[_row_spec(d_out)],
        out_shape=[jax.ShapeDtypeStruct((n, d_out), jnp.float32)],
    )(P0, P1, x5, dinv, Wp, bp)[0]


def kernel(features, edges, weights, W1, b1, W2, b2, W3, b3, W4, b4):
    n = features.shape[0]
    e_cnt = edges.shape[1]
    row = edges[0].astype(jnp.int32)
    col = edges[1].astype(jnp.int32)
    w = weights.astype(jnp.float32)

    T = -(-e_cnt // (NW * C * 2 * K)) * (2 * K)
    e_pad = NW * T * C
    npad = e_pad - e_cnt
    if npad:
        fill = (jnp.arange(npad, dtype=jnp.int32) * 97) % n
        row = jnp.concatenate([row, fill])
        col = jnp.concatenate([col, fill])
        w = jnp.concatenate([w, jnp.zeros((npad,), jnp.float32)])
    row_r = row.reshape(NW, T, C)
    col_r = col.reshape(NW, T, C)
    w_r = w.reshape(NW, T, C)

    n_pad = -(-n // (NS * C)) * (NS * C)
    degp = _sc_deg(col_r, w_r, n_pad)
    p0 = degp[0, :n, None]
    p1 = degp[1, :n, None]

    def padw(W, a, b):
        return jnp.pad(W, ((0, a - W.shape[0]), (0, b - W.shape[1])))

    W1p = padw(W1, 128, 32)
    W2p = padw(W2, 32, 32)
    W3p = padw(W3, 32, 16)
    I16 = jnp.eye(16, dtype=jnp.float32)
    W4p = padw(W4, 16, 48)
    b1p = jnp.pad(b1, (0, 12))[None, :]
    b2p = jnp.pad(b2, (0, 12))[None, :]
    b3p = jnp.pad(b3, (0, 6))[None, :]
    b4p = b4[None, :]

    def agg(hd, d_pad):
        P = _sc_agg(hd, row_r, col_r, w_r, d_pad, n_pad)
        return P[0, :n], P[1, :n]

    hd1, h1, dinv = _tc_pre1(features, W1p, p0, p1)
    S0, S1 = agg(hd1, 32)
    h2, hd2 = _tc_step(S0, S1, h1, dinv, b1p, W2p)
    S0, S1 = agg(hd2, 32)
    h3, hd3 = _tc_step(S0, S1, h2, dinv, b2p, W2p)
    S0, S1 = agg(hd3, 32)
    h4, hd4 = _tc_step(S0, S1, h3, dinv, b2p, W3p)
    S0, S1 = agg(hd4, 16)
    x5, xd5 = _tc_step(S0, S1, h4, dinv, b3p, I16)
    S0, S1 = agg(xd5, 16)
    return _tc_final(S0, S1, x5, dinv, W4p, b4p, 40)


# speedup vs baseline: 1.0114x; 1.0114x over previous
import functools

import jax
import jax.numpy as jnp
from jax import lax
from jax.experimental import pallas as pl
from jax.experimental.pallas import tpu as pltpu, tpu_sc as plsc

NC, NS, L = 2, 16, 16
NW = NC * NS
C = 128
K = 4

_SC_PARAMS = pltpu.CompilerParams(
    needs_layout_passes=False, use_tc_tiling_on_sc=False
)


def _sc_deg(col_r, w_r, n_pad):
    T = col_r.shape[1]
    rows_per_tile = n_pad // NS
    mesh = plsc.VectorSubcoreMesh(core_axis_name="c", subcore_axis_name="s")

    @functools.partial(
        pl.kernel,
        mesh=mesh,
        out_type=jax.ShapeDtypeStruct((NC, n_pad), jnp.float32),
        scratch_types=[
            pltpu.VMEM((T, C), jnp.int32),
            pltpu.VMEM((T, C), jnp.float32),
            pltpu.VMEM((rows_per_tile,), jnp.float32),
            pltpu.VMEM_SHARED((n_pad,), jnp.float32),
            pltpu.SemaphoreType.DMA,
        ],
        compiler_params=_SC_PARAMS,
    )
    def k(col_hbm, w_hbm, out_hbm, col_v, w_v, zbuf, acc_sh, sem):
        c = lax.axis_index("c")
        s = lax.axis_index("s")
        wid = s * NC + c
        pltpu.sync_copy(col_hbm.at[wid], col_v)
        pltpu.sync_copy(w_hbm.at[wid], w_v)

        @pl.loop(0, rows_per_tile // L)
        def _zero(i):
            zbuf[pl.ds(i * L, L)] = jnp.zeros((L,), jnp.float32)

        pltpu.sync_copy(zbuf, acc_sh.at[pl.ds(s * rows_per_tile, rows_per_tile)])
        plsc.subcore_barrier()

        @pl.loop(0, T)
        def _fire(j):
            pltpu.async_copy(w_v.at[j], acc_sh.at[col_v.at[j]], sem, add=True)

        @pl.loop(0, T)
        def _drain(j):
            pltpu.make_async_copy(w_v.at[j], acc_sh.at[col_v.at[j]], sem).wait()

        plsc.subcore_barrier()
        pltpu.sync_copy(
            acc_sh.at[pl.ds(s * rows_per_tile, rows_per_tile)],
            out_hbm.at[c, pl.ds(s * rows_per_tile, rows_per_tile)],
        )

    return k(col_r, w_r)


def _sc_agg(hd, row_r, col_r, w_r, d_pad, n_pad):
    T = row_r.shape[1]
    rows_per_tile = n_pad // NS
    nz = rows_per_tile // C
    R = 2 * K
    mesh = plsc.VectorSubcoreMesh(core_axis_name="c", subcore_axis_name="s")

    @functools.partial(
        pl.kernel,
        mesh=mesh,
        out_type=jax.ShapeDtypeStruct((NC, n_pad, d_pad), jnp.float32),
        scratch_types=[
            pltpu.VMEM((T, C), jnp.int32),
            pltpu.VMEM((T, C), jnp.int32),
            pltpu.VMEM((T, C), jnp.float32),
            pltpu.VMEM((R, C, d_pad), jnp.float32),
            pltpu.VMEM_SHARED((n_pad, d_pad), jnp.float32),
        ] + [pltpu.SemaphoreType.DMA] * (2 * R),
        compiler_params=_SC_PARAMS,
    )
    def k(hd_hbm, row_hbm, col_hbm, w_hbm, out_hbm,
          row_v, col_v, w_v, msg_v, acc_sh, *sems):
        gsem = sems[:R]
        ssem = sems[R:]
        c = lax.axis_index("c")
        s = lax.axis_index("s")
        wid = s * NC + c
        pltpu.sync_copy(row_hbm.at[wid], row_v)
        pltpu.sync_copy(col_hbm.at[wid], col_v)
        pltpu.sync_copy(w_hbm.at[wid], w_v)

        zb = msg_v.at[0]

        @pl.loop(0, (C * d_pad) // L)
        def _zero(i):
            r = i // (d_pad // L)
            kk = i % (d_pad // L)
            zb[r, pl.ds(kk * L, L)] = jnp.zeros((L,), jnp.float32)

        @pl.loop(0, nz)
        def _zcopy(kz):
            pltpu.sync_copy(zb, acc_sh.at[pl.ds(s * rows_per_tile + kz * C, C)])

        plsc.subcore_barrier()

        for b in range(K):
            pltpu.async_copy(hd_hbm.at[row_v.at[b]], msg_v.at[b], gsem[b])

        def scale_chunk(mb, j):
            @pl.loop(0, C // 16)
            def _grp(q):
                wrow = w_v[j, pl.ds(q * 16, 16)]
                for l in range(16):
                    wv = jnp.full((L,), wrow[l], jnp.float32)
                    e = q * 16 + l
                    for kk in range(d_pad // L):
                        sl = pl.ds(kk * L, L)
                        mb[e, sl] = mb[e, sl] * wv

        @pl.loop(0, T // R)
        def _ring(gi):
            for v in range(R):
                j = gi * R + v
                mb = msg_v.at[v]
                pltpu.make_async_copy(hd_hbm.at[row_v.at[j]], mb, gsem[v]).wait()
                scale_chunk(mb, j)
                pltpu.async_copy(mb, acc_sh.at[col_v.at[j]], ssem[v], add=True)

                jg = j + K
                bg = (v + K) % R
                mg = msg_v.at[bg]

                @pl.when(jg < T)
                def _refill():
                    @pl.when(jg >= R)
                    def _drain_prev_scatter():
                        pltpu.make_async_copy(
                            mg, acc_sh.at[col_v.at[jg - R]], ssem[bg]
                        ).wait()

                    pltpu.async_copy(hd_hbm.at[row_v.at[jg]], mg, gsem[bg])

        for b in range(R):
            pltpu.make_async_copy(
                msg_v.at[b], acc_sh.at[col_v.at[T - R + b]], ssem[b]
            ).wait()

        plsc.subcore_barrier()
        pltpu.sync_copy(
            acc_sh.at[pl.ds(s * rows_per_tile, rows_per_tile)],
            out_hbm.at[c, pl.ds(s * rows_per_tile, rows_per_tile)],
        )

    return k(hd, row_r, col_r, w_r)


def _pad_d(x, d_pad):
    d = x.shape[1]
    if d == d_pad:
        return x
    return jnp.pad(x, ((0, 0), (0, d_pad - d)))


_RB = 10000


def _row_spec(d):
    return pl.BlockSpec((_RB, d), lambda r: (r, 0))


def _full_spec(a, b):
    return pl.BlockSpec((a, b), lambda r: (0, 0))


def _tc_pre1(x, Wp, p0, p1):
    n, d_in = x.shape
    d_o = Wp.shape[1]

    def body(x_ref, w_ref, p0_ref, p1_ref, hd_ref, h_ref, dinv_ref):
        dinv = lax.rsqrt(1.0 + p0_ref[...] + p1_ref[...])
        h = jnp.dot(x_ref[...], w_ref[...], preferred_element_type=jnp.float32)
        h_ref[...] = h
        hd_ref[...] = h * dinv
        dinv_ref[...] = dinv

    return pl.pallas_call(
        body,
        grid=(n // _RB,),
        in_specs=[_row_spec(d_in), _full_spec(d_in, d_o),
                  _row_spec(1), _row_spec(1)],
        out_specs=[_row_spec(d_o), _row_spec(d_o), _row_spec(1)],
        out_shape=[jax.ShapeDtypeStruct((n, d_o), jnp.float32),
                   jax.ShapeDtypeStruct((n, d_o), jnp.float32),
                   jax.ShapeDtypeStruct((n, 1), jnp.float32)],
    )(x, Wp, p0, p1)


def _tc_step(P0, P1, h, dinv, bp, Wp):
    n, d_prev = h.shape
    d_next = Wp.shape[1]

    def body(p0_ref, p1_ref, h_ref, dinv_ref, b_ref, w_ref, hn_ref, hdn_ref):
        dinv = dinv_ref[...]
        s = p0_ref[...] + p1_ref[...]
        out = dinv * s + (dinv * dinv) * h_ref[...] + b_ref[...]
        xn = jnp.maximum(out, 0.0)
        hn = jnp.dot(xn, w_ref[...], preferred_element_type=jnp.float32)
        hn_ref[...] = hn
        hdn_ref[...] = hn * dinv

    return pl.pallas_call(
        body,
        grid=(n // _RB,),
        in_specs=[_row_spec(d_prev), _row_spec(d_prev), _row_spec(d_prev),
                  _row_spec(1), _full_spec(1, d_prev),
                  _full_spec(d_prev, d_next)],
        out_specs=[_row_spec(d_next), _row_spec(d_next)],
        out_shape=[jax.ShapeDtypeStruct((n, d_next), jnp.float32),
                   jax.ShapeDtypeStruct((n, d_next), jnp.float32)],
    )(P0, P1, h, dinv, bp, Wp)


def _tc_final(P0, P1, x5, dinv, Wp, bp, d_out):
    n, d_prev = x5.shape
    d_op = Wp.shape[1]

    def body(p0_ref, p1_ref, x_ref, dinv_ref, w_ref, b_ref, o_ref):
        dinv = dinv_ref[...]
        s = p0_ref[...] + p1_ref[...]
        ax = dinv * s + (dinv * dinv) * x_ref[...]
        out = jnp.dot(ax, w_ref[...], preferred_element_type=jnp.float32)
        out = out[:, :d_out] + b_ref[...]
        m = jnp.max(out, axis=1, keepdims=True)
        z = out - m
        lse = jnp.log(jnp.sum(jnp.exp(z), axis=1, keepdims=True))
        o_ref[...] = z - lse

    return pl.pallas_call(
        body,
        grid=(n // _RB,),
        in_specs=[_row_spec(d_prev), _row_spec(d_prev), _row_spec(d_prev),
                  _row_spec(1), _full_spec(d_prev, d_op),
                  _full_spec(1, d_out)],
        out_specs=[_row_spec(d_out)],
        out_shape=[jax.ShapeDtypeStruct((n, d_out), jnp.float32)],
    )(P0, P1, x5, dinv, Wp, bp)[0]


def kernel(features, edges, weights, W1, b1, W2, b2, W3, b3, W4, b4):
    n = features.shape[0]
    e_cnt = edges.shape[1]
    row = edges[0].astype(jnp.int32)
    col = edges[1].astype(jnp.int32)
    w = weights.astype(jnp.float32)

    T = -(-e_cnt // (NW * C * 2 * K)) * (2 * K)
    e_pad = NW * T * C
    npad = e_pad - e_cnt
    if npad:
        fill = (jnp.arange(npad, dtype=jnp.int32) * 97) % n
        row = jnp.concatenate([row, fill])
        col = jnp.concatenate([col, fill])
        w = jnp.concatenate([w, jnp.zeros((npad,), jnp.float32)])
    row_r = row.reshape(NW, T, C)
    col_r = col.reshape(NW, T, C)
    w_r = w.reshape(NW, T, C)

    n_pad = -(-n // (NS * C)) * (NS * C)
    degp = _sc_deg(col_r, w_r, n_pad)
    p0 = degp[0, :n, None]
    p1 = degp[1, :n, None]

    def padw(W, a, b):
        return jnp.pad(W, ((0, a - W.shape[0]), (0, b - W.shape[1])))

    W1p = padw(W1, 128, 32)
    W2p = padw(W2, 32, 32)
    W3p = padw(W3, 32, 16)
    I16 = jnp.eye(16, dtype=jnp.float32)
    W4p = padw(W4, 16, 48)
    b1p = jnp.pad(b1, (0, 12))[None, :]
    b2p = jnp.pad(b2, (0, 12))[None, :]
    b3p = jnp.pad(b3, (0, 6))[None, :]
    b4p = b4[None, :]

    def agg(hd, d_pad):
        P = _sc_agg(hd, row_r, col_r, w_r, d_pad, n_pad)
        return P[0, :n], P[1, :n]

    hd1, h1, dinv = _tc_pre1(features, W1p, p0, p1)
    S0, S1 = agg(hd1, 32)
    h2, hd2 = _tc_step(S0, S1, h1, dinv, b1p, W2p)
    S0, S1 = agg(hd2, 32)
    h3, hd3 = _tc_step(S0, S1, h2, dinv, b2p, W2p)
    S0, S1 = agg(hd3, 32)
    h4, hd4 = _tc_step(S0, S1, h3, dinv, b2p, W3p)
    S0, S1 = agg(hd4, 16)
    x5, xd5 = _tc_step(S0, S1, h4, dinv, b3p, I16)
    S0, S1 = agg(xd5, 16)
    return _tc_final(S0, S1, x5, dinv, W4p, b4p, 40)
